# baseline (speedup 1.0000x reference)
import functools

import jax
import jax.numpy as jnp
from jax import lax
from jax.experimental import pallas as pl
from jax.experimental.pallas import tpu as pltpu
from jax.experimental.pallas import tpu_sc as plsc

B = 16384
MF_DIM = 16
MLP_DIM = 32

NC, NS = 2, 16
NW = NC * NS
BPW = B // NW
CHUNK = 128
NCHUNK = BPW // CHUNK


def _sc_gather_body(user3, item3, mf_u_t, mf_i_t, ml_u_t, ml_i_t,
                    mf_out, mlu_out, mli_out,
                    uidx, iidx, bmfu, bmfi, bmlu, bmli, sem_mf, sem_ml):
    wid = lax.axis_index("s") * NC + lax.axis_index("c")
    base = wid * BPW
    pltpu.sync_copy(user3.at[wid], uidx)
    pltpu.sync_copy(item3.at[wid], iidx)

    handles_mf = []
    handles_ml = []
    for j in range(NCHUNK):
        sl = pl.ds(j * CHUNK, CHUNK)
        handles_mf.append(pltpu.async_copy(mf_u_t.at[uidx.at[j]], bmfu.at[sl], sem_mf))
        handles_mf.append(pltpu.async_copy(mf_i_t.at[iidx.at[j]], bmfi.at[sl], sem_mf))
        handles_ml.append(pltpu.async_copy(ml_u_t.at[uidx.at[j]], bmlu.at[sl], sem_ml))
        handles_ml.append(pltpu.async_copy(ml_i_t.at[iidx.at[j]], bmli.at[sl], sem_ml))
    for h in handles_mf:
        h.wait()

    def mul_row(i, _):
        bmfu[i] = bmfu[i] * bmfi[i]
        return 0
    lax.fori_loop(0, BPW, mul_row, 0, unroll=8)

    pltpu.sync_copy(bmfu, mf_out.at[pl.ds(base, BPW)])
    for h in handles_ml:
        h.wait()
    pltpu.sync_copy(bmlu, mlu_out.at[pl.ds(base, BPW)])
    pltpu.sync_copy(bmli, mli_out.at[pl.ds(base, BPW)])


_sc_gather = functools.partial(
    pl.kernel,
    out_type=[
        jax.ShapeDtypeStruct((B, MF_DIM), jnp.float32),
        jax.ShapeDtypeStruct((B, MLP_DIM), jnp.float32),
        jax.ShapeDtypeStruct((B, MLP_DIM), jnp.float32),
    ],
    mesh=plsc.VectorSubcoreMesh(core_axis_name="c", subcore_axis_name="s"),
    compiler_params=pltpu.CompilerParams(use_tc_tiling_on_sc=False),
    scratch_types=[
        pltpu.VMEM((NCHUNK, CHUNK), jnp.int32),
        pltpu.VMEM((NCHUNK, CHUNK), jnp.int32),
        pltpu.VMEM((BPW, MF_DIM), jnp.float32),
        pltpu.VMEM((BPW, MF_DIM), jnp.float32),
        pltpu.VMEM((BPW, MLP_DIM), jnp.float32),
        pltpu.VMEM((BPW, MLP_DIM), jnp.float32),
        pltpu.SemaphoreType.DMA,
        pltpu.SemaphoreType.DMA,
    ],
)(_sc_gather_body)


B_TC = 2048


def _mlp_body(mf_ref, u_ref, i_ref, w1u_ref, w1i_ref, b1_ref, w2_ref, b2_ref,
              woa_ref, wob_ref, bo_ref, out_ref):
    h1 = jnp.dot(u_ref[...], w1u_ref[...], preferred_element_type=jnp.float32)
    h1 += jnp.dot(i_ref[...], w1i_ref[...], preferred_element_type=jnp.float32)
    h1 = jnp.maximum(h1 + b1_ref[...], 0.0)
    h2 = jnp.dot(h1, w2_ref[...], preferred_element_type=jnp.float32)
    h2 = jnp.maximum(h2 + b2_ref[...], 0.0)
    out = jnp.dot(mf_ref[...], woa_ref[...], preferred_element_type=jnp.float32)
    out += jnp.dot(h2, wob_ref[...], preferred_element_type=jnp.float32)
    out_ref[...] = out + bo_ref[...]


def _mlp(mf_out, mlu, mli, w1u, w1i, b1, w2, b2, woa, wob, bo):
    grid = (B // B_TC,)
    full = lambda r, c: pl.BlockSpec((r, c), lambda i: (0, 0))
    return pl.pallas_call(
        _mlp_body,
        grid=grid,
        in_specs=[
            pl.BlockSpec((B_TC, MF_DIM), lambda i: (i, 0)),
            pl.BlockSpec((B_TC, MLP_DIM), lambda i: (i, 0)),
            pl.BlockSpec((B_TC, MLP_DIM), lambda i: (i, 0)),
            full(MLP_DIM, 64),
            full(MLP_DIM, 64),
            full(1, 64),
            full(64, MLP_DIM),
            full(1, MLP_DIM),
            full(MF_DIM, 1),
            full(MLP_DIM, 1),
            full(1, 1),
        ],
        out_specs=pl.BlockSpec((B_TC, 1), lambda i: (i, 0)),
        out_shape=jax.ShapeDtypeStruct((B, 1), jnp.float32),
    )(mf_out, mlu, mli, w1u, w1i, b1, w2, b2, woa, wob, bo)


def kernel(user, item, mf_user_t, mf_item_t, mlp_user_t, mlp_item_t,
           W1, b1, W2, b2, Wo, bo):
    user3 = user.reshape(NW, NCHUNK, CHUNK)
    item3 = item.reshape(NW, NCHUNK, CHUNK)
    mf_out, mlu, mli = _sc_gather(user3, item3, mf_user_t, mf_item_t,
                                  mlp_user_t, mlp_item_t)
    w1u = W1[:, :MLP_DIM].T
    w1i = W1[:, MLP_DIM:].T
    woa = Wo[:, :MF_DIM].T
    wob = Wo[:, MF_DIM:].T
    return _mlp(mf_out, mlu, mli, w1u, w1i, b1[None, :], W2.T,
                b2[None, :], woa, wob, bo[None, :])
